# baseline (speedup 1.0000x reference)
import functools

import jax
import jax.numpy as jnp
from jax import lax
from jax.experimental import pallas as pl
from jax.experimental.pallas import tpu as pltpu
from jax.experimental.pallas import tpu_sc as plsc

_NUM_CODEBOOK = 8192
_EMBED_DIM = 256
_BETA = 0.25
_TOKENS = 1024
_BATCH = 8
_TK = 1368
_NK = 6
_TILES_PER_WIN = 2
_EPS = 1e-12

_SC_CORES = 2
_SC_SUBCORES = 16
_SC_WORKERS = _SC_CORES * _SC_SUBCORES
_ROWS_PER_WORKER = _NUM_CODEBOOK // _SC_WORKERS


def _vq_body(z_ref, cb_ref, idx_ref,
             zn_s, znsq_s, wmin_s, widx_s, carry_s, fidx_s):
    j = pl.program_id(1)

    @pl.when(j == 0)
    def _init():
        zb = z_ref[0]
        norm = jnp.sqrt(jnp.sum(zb * zb, axis=0, keepdims=True))
        zn = zb / jnp.maximum(norm, _EPS)
        zn_s[...] = zn
        znsq_s[...] = jnp.sum(zn * zn, axis=0, keepdims=True)
        wmin_s[...] = jnp.full((1, _TOKENS), jnp.inf, jnp.float32)
        carry_s[...] = jnp.full((1, _TOKENS), jnp.inf, jnp.float32)

    cb = cb_ref[...]
    cn2_raw = jnp.sum(cb * cb, axis=1, keepdims=True)
    normc = jnp.sqrt(cn2_raw)
    cbn = cb / jnp.maximum(normc, _EPS)
    cbsq = jnp.sum(cbn * cbn, axis=1, keepdims=True)

    dots = lax.dot_general(
        cbn, zn_s[...], (((1,), (0,)), ((), ())),
        preferred_element_type=jnp.float32,
        precision=lax.Precision.DEFAULT)
    dist = (znsq_s[...] + cbsq) - 2.0 * dots

    gidx = lax.broadcasted_iota(jnp.int32, (_TK, _TOKENS), 0) + j * _TK
    dist = jnp.where(gidx < _NUM_CODEBOOK, dist, jnp.float32(jnp.inf))
    lmin = jnp.min(dist, axis=0, keepdims=True)
    larg = jnp.min(jnp.where(dist == lmin, gidx, jnp.int32(2 ** 30)),
                   axis=0, keepdims=True)

    better = lmin < wmin_s[...]
    wmin = jnp.where(better, lmin, wmin_s[...])
    widx = jnp.where(better, larg, widx_s[...])
    wmin_s[...] = wmin
    widx_s[...] = widx

    @pl.when(j % _TILES_PER_WIN == _TILES_PER_WIN - 1)
    def _window_end():
        take = wmin < carry_s[...]
        carry_s[...] = jnp.where(
            take, wmin.astype(jnp.bfloat16).astype(jnp.float32), carry_s[...])
        fidx = jnp.where(take, widx, fidx_s[...])
        fidx_s[...] = fidx
        wmin_s[...] = jnp.full((1, _TOKENS), jnp.inf, jnp.float32)

        @pl.when(j == _NK - 1)
        def _emit():
            idx_ref[...] = fidx.reshape(1, 1, _TOKENS)


def _vq_argmin(z3, codebook):
    return pl.pallas_call(
        _vq_body,
        grid=(_BATCH, _NK),
        in_specs=[
            pl.BlockSpec((1, _EMBED_DIM, _TOKENS), lambda b, j: (b, 0, 0)),
            pl.BlockSpec((_TK, _EMBED_DIM), lambda b, j: (j, 0)),
        ],
        out_specs=pl.BlockSpec((1, 1, _TOKENS), lambda b, j: (b, 0, 0)),
        out_shape=jax.ShapeDtypeStruct((_BATCH, 1, _TOKENS), jnp.int32),
        scratch_shapes=[
            pltpu.VMEM((_EMBED_DIM, _TOKENS), jnp.float32),
            pltpu.VMEM((1, _TOKENS), jnp.float32),
            pltpu.VMEM((1, _TOKENS), jnp.float32),
            pltpu.VMEM((1, _TOKENS), jnp.int32),
            pltpu.VMEM((1, _TOKENS), jnp.float32),
            pltpu.VMEM((1, _TOKENS), jnp.int32),
        ],
        compiler_params=pltpu.CompilerParams(
            dimension_semantics=("parallel", "arbitrary")),
    )(z3, codebook)


def _gather_body(table_hbm, idx_hbm, out_hbm, idx_v, rows_v, sem):
    wid = lax.axis_index("s") * _SC_CORES + lax.axis_index("c")
    base = wid * _ROWS_PER_WORKER
    pltpu.sync_copy(idx_hbm.at[pl.ds(base, _ROWS_PER_WORKER)], idx_v)
    pltpu.async_copy(table_hbm.at[idx_v], rows_v, sem).wait()
    pltpu.sync_copy(rows_v, out_hbm.at[pl.ds(base, _ROWS_PER_WORKER)])


@functools.cache
def _sc_gather():
    return pl.kernel(
        _gather_body,
        out_type=jax.ShapeDtypeStruct((_NUM_CODEBOOK, _EMBED_DIM),
                                      jnp.float32),
        mesh=plsc.VectorSubcoreMesh(
            core_axis_name="c", subcore_axis_name="s",
            num_cores=_SC_CORES, num_subcores=_SC_SUBCORES),
        scratch_types=[
            pltpu.VMEM((_ROWS_PER_WORKER,), jnp.int32),
            pltpu.VMEM((_ROWS_PER_WORKER, _EMBED_DIM), jnp.float32),
            pltpu.SemaphoreType.DMA,
        ],
    )


_LCHUNK = 1024


def _loss_body(zf_ref, rows_ref, cl_ref, commit_ref, loss_ref, acc_s):
    i = pl.program_id(0)
    zb = zf_ref[...]
    norm = jnp.sqrt(jnp.sum(zb * zb, axis=1, keepdims=True))
    zn = zb / jnp.maximum(norm, _EPS)
    d = rows_ref[...] - zn
    s = jnp.sum(d * d)
    prev = jnp.where(i == 0, 0.0, acc_s[0, 0])
    total = prev + s
    acc_s[0, 0] = total

    @pl.when(i == pl.num_programs(0) - 1)
    def _emit():
        cl = total / jnp.float32(_BATCH * _TOKENS * _EMBED_DIM)
        cl_ref[...] = jnp.full((1, 1), cl, jnp.float32)
        commit_ref[...] = jnp.full((1, 1), cl, jnp.float32)
        loss_ref[...] = jnp.full((1, 1), cl + _BETA * cl, jnp.float32)


def _vq_loss(z_flat, rows):
    n = _BATCH * _TOKENS
    return pl.pallas_call(
        _loss_body,
        grid=(n // _LCHUNK,),
        in_specs=[
            pl.BlockSpec((_LCHUNK, _EMBED_DIM), lambda i: (i, 0)),
            pl.BlockSpec((_LCHUNK, _EMBED_DIM), lambda i: (i, 0)),
        ],
        out_specs=[
            pl.BlockSpec((1, 1), lambda i: (0, 0)),
            pl.BlockSpec((1, 1), lambda i: (0, 0)),
            pl.BlockSpec((1, 1), lambda i: (0, 0)),
        ],
        out_shape=[
            jax.ShapeDtypeStruct((1, 1), jnp.float32),
            jax.ShapeDtypeStruct((1, 1), jnp.float32),
            jax.ShapeDtypeStruct((1, 1), jnp.float32),
        ],
        scratch_shapes=[pltpu.SMEM((1, 1), jnp.float32)],
        compiler_params=pltpu.CompilerParams(
            dimension_semantics=("arbitrary",)),
    )(z_flat, rows)


def kernel(z, codebook):
    b, d, h, w = z.shape
    z3 = z.reshape(b, d, h * w)
    idx = _vq_argmin(z3, codebook)
    rows = _sc_gather()(codebook, idx.reshape(-1))
    z_flat = jnp.transpose(z3, (0, 2, 1)).reshape(-1, d)
    cl, commit, loss = _vq_loss(z_flat, rows)
    q = jnp.transpose(rows.reshape(b, h, w, d), (0, 3, 1, 2))
    return (q, loss[0, 0], cl[0, 0], commit[0, 0])


# speedup vs baseline: 1.1508x; 1.1508x over previous
import functools

import jax
import jax.numpy as jnp
from jax import lax
from jax.experimental import pallas as pl
from jax.experimental.pallas import tpu as pltpu
from jax.experimental.pallas import tpu_sc as plsc

_NUM_CODEBOOK = 8192
_EMBED_DIM = 256
_BETA = 0.25
_TOKENS = 1024
_BATCH = 8
_TK = 1368
_NK = 6
_TILES_PER_WIN = 2
_EPS = 1e-12

_SC_CORES = 2
_SC_SUBCORES = 16
_SC_WORKERS = _SC_CORES * _SC_SUBCORES
_ROWS_PER_WORKER = _NUM_CODEBOOK // _SC_WORKERS


def _prep_body(cb_ref, cbn_ref, cbsq_ref):
    cb = cb_ref[...]
    cn2 = jnp.sum(cb * cb, axis=1, keepdims=True)
    normc = jnp.sqrt(cn2)
    cbn = cb / jnp.maximum(normc, _EPS)
    cbn_ref[...] = cbn
    cbsq_ref[...] = jnp.sum(cbn * cbn, axis=1, keepdims=True)


def _prep(codebook):
    return pl.pallas_call(
        _prep_body,
        grid=(_NK,),
        in_specs=[pl.BlockSpec((_TK, _EMBED_DIM), lambda j: (j, 0))],
        out_specs=[
            pl.BlockSpec((_TK, _EMBED_DIM), lambda j: (j, 0)),
            pl.BlockSpec((_TK, 1), lambda j: (j, 0)),
        ],
        out_shape=[
            jax.ShapeDtypeStruct((_NUM_CODEBOOK, _EMBED_DIM), jnp.float32),
            jax.ShapeDtypeStruct((_NK * _TK, 1), jnp.float32),
        ],
        compiler_params=pltpu.CompilerParams(
            dimension_semantics=("arbitrary",)),
    )(codebook)


def _vq_body(z_ref, cbn_ref, cbsq_ref, idx_ref,
             zn_s, znsq_s, wmin_s, widx_s, carry_s, fidx_s):
    j = pl.program_id(1)

    @pl.when(j == 0)
    def _init():
        zb = z_ref[0]
        norm = jnp.sqrt(jnp.sum(zb * zb, axis=0, keepdims=True))
        zn = zb / jnp.maximum(norm, _EPS)
        zn_s[...] = zn
        znsq_s[...] = jnp.sum(zn * zn, axis=0, keepdims=True)
        wmin_s[...] = jnp.full((1, _TOKENS), jnp.inf, jnp.float32)
        carry_s[...] = jnp.full((1, _TOKENS), jnp.inf, jnp.float32)

    dots = lax.dot_general(
        cbn_ref[...], zn_s[...], (((1,), (0,)), ((), ())),
        preferred_element_type=jnp.float32,
        precision=lax.Precision.DEFAULT)
    dist = (znsq_s[...] + cbsq_ref[...]) - 2.0 * dots

    _OFF = 0x4B000000
    liota = lax.broadcasted_iota(jnp.int32, (_TK, _TOKENS), 0) + _OFF
    _BIG = jnp.int32(_OFF + 0x4000)

    def _update(dist):
        lmin = jnp.min(dist, axis=0, keepdims=True)
        cand = lax.bitcast_convert_type(
            jnp.where(dist == lmin, liota, _BIG), jnp.float32)
        lloc = jnp.min(cand, axis=0, keepdims=True)
        larg = (lax.bitcast_convert_type(lloc, jnp.int32) - _OFF
                + j * _TK).astype(jnp.float32)
        better = lmin < wmin_s[...]
        wmin = jnp.where(better, lmin, wmin_s[...])
        widx = jnp.where(better, larg, widx_s[...])
        wmin_s[...] = wmin
        widx_s[...] = widx

        @pl.when(j % _TILES_PER_WIN == _TILES_PER_WIN - 1)
        def _window_end():
            take = wmin < carry_s[...]
            carry_s[...] = jnp.where(
                take, wmin.astype(jnp.bfloat16).astype(jnp.float32),
                carry_s[...])
            fidx = jnp.where(take, widx, fidx_s[...])
            fidx_s[...] = fidx

            @pl.when(j == _NK - 1)
            def _emit():
                idx_ref[...] = fidx.astype(jnp.int32).reshape(1, 1, _TOKENS)

            @pl.when(j < _NK - 1)
            def _reset():
                wmin_s[...] = jnp.full((1, _TOKENS), jnp.inf, jnp.float32)

    @pl.when(j < _NK - 1)
    def _plain():
        _update(dist)

    @pl.when(j == _NK - 1)
    def _masked():
        gidx = lax.broadcasted_iota(jnp.int32, (_TK, _TOKENS), 0) + j * _TK
        _update(jnp.where(gidx < _NUM_CODEBOOK, dist, jnp.float32(jnp.inf)))


def _vq_argmin(z3, cbn, cbsq):
    return pl.pallas_call(
        _vq_body,
        grid=(_BATCH, _NK),
        in_specs=[
            pl.BlockSpec((1, _EMBED_DIM, _TOKENS), lambda b, j: (b, 0, 0)),
            pl.BlockSpec((_TK, _EMBED_DIM), lambda b, j: (j, 0)),
            pl.BlockSpec((_TK, 1), lambda b, j: (j, 0)),
        ],
        out_specs=pl.BlockSpec((1, 1, _TOKENS), lambda b, j: (b, 0, 0)),
        out_shape=jax.ShapeDtypeStruct((_BATCH, 1, _TOKENS), jnp.int32),
        scratch_shapes=[
            pltpu.VMEM((_EMBED_DIM, _TOKENS), jnp.float32),
            pltpu.VMEM((1, _TOKENS), jnp.float32),
            pltpu.VMEM((1, _TOKENS), jnp.float32),
            pltpu.VMEM((1, _TOKENS), jnp.float32),
            pltpu.VMEM((1, _TOKENS), jnp.float32),
            pltpu.VMEM((1, _TOKENS), jnp.float32),
        ],
        compiler_params=pltpu.CompilerParams(
            dimension_semantics=("parallel", "arbitrary")),
    )(z3, cbn, cbsq)


def _gather_body(table_hbm, idx_hbm, out_hbm, idx_v, rows_v, sem):
    wid = lax.axis_index("s") * _SC_CORES + lax.axis_index("c")
    base = wid * _ROWS_PER_WORKER
    pltpu.sync_copy(idx_hbm.at[pl.ds(base, _ROWS_PER_WORKER)], idx_v)
    pltpu.async_copy(table_hbm.at[idx_v], rows_v, sem).wait()
    pltpu.sync_copy(rows_v, out_hbm.at[pl.ds(base, _ROWS_PER_WORKER)])


@functools.cache
def _sc_gather():
    return pl.kernel(
        _gather_body,
        out_type=jax.ShapeDtypeStruct((_NUM_CODEBOOK, _EMBED_DIM),
                                      jnp.float32),
        mesh=plsc.VectorSubcoreMesh(
            core_axis_name="c", subcore_axis_name="s",
            num_cores=_SC_CORES, num_subcores=_SC_SUBCORES),
        scratch_types=[
            pltpu.VMEM((_ROWS_PER_WORKER,), jnp.int32),
            pltpu.VMEM((_ROWS_PER_WORKER, _EMBED_DIM), jnp.float32),
            pltpu.SemaphoreType.DMA,
        ],
    )


_LCHUNK = 1024


def _loss_body(zf_ref, rows_ref, cl_ref, commit_ref, loss_ref, acc_s):
    i = pl.program_id(0)
    zb = zf_ref[...]
    norm = jnp.sqrt(jnp.sum(zb * zb, axis=1, keepdims=True))
    zn = zb / jnp.maximum(norm, _EPS)
    d = rows_ref[...] - zn
    s = jnp.sum(d * d)
    prev = jnp.where(i == 0, 0.0, acc_s[0, 0])
    total = prev + s
    acc_s[0, 0] = total

    @pl.when(i == pl.num_programs(0) - 1)
    def _emit():
        cl = total / jnp.float32(_BATCH * _TOKENS * _EMBED_DIM)
        cl_ref[...] = jnp.full((1, 1), cl, jnp.float32)
        commit_ref[...] = jnp.full((1, 1), cl, jnp.float32)
        loss_ref[...] = jnp.full((1, 1), cl + _BETA * cl, jnp.float32)


def _vq_loss(z_flat, rows):
    n = _BATCH * _TOKENS
    return pl.pallas_call(
        _loss_body,
        grid=(n // _LCHUNK,),
        in_specs=[
            pl.BlockSpec((_LCHUNK, _EMBED_DIM), lambda i: (i, 0)),
            pl.BlockSpec((_LCHUNK, _EMBED_DIM), lambda i: (i, 0)),
        ],
        out_specs=[
            pl.BlockSpec((1, 1), lambda i: (0, 0)),
            pl.BlockSpec((1, 1), lambda i: (0, 0)),
            pl.BlockSpec((1, 1), lambda i: (0, 0)),
        ],
        out_shape=[
            jax.ShapeDtypeStruct((1, 1), jnp.float32),
            jax.ShapeDtypeStruct((1, 1), jnp.float32),
            jax.ShapeDtypeStruct((1, 1), jnp.float32),
        ],
        scratch_shapes=[pltpu.SMEM((1, 1), jnp.float32)],
        compiler_params=pltpu.CompilerParams(
            dimension_semantics=("arbitrary",)),
    )(z_flat, rows)


def kernel(z, codebook):
    b, d, h, w = z.shape
    z3 = z.reshape(b, d, h * w)
    cbn, cbsq = _prep(codebook)
    idx = _vq_argmin(z3, cbn, cbsq)
    rows = _sc_gather()(codebook, idx.reshape(-1))
    z_flat = jnp.transpose(z3, (0, 2, 1)).reshape(-1, d)
    cl, commit, loss = _vq_loss(z_flat, rows)
    q = jnp.transpose(rows.reshape(b, h, w, d), (0, 3, 1, 2))
    return (q, loss[0, 0], cl[0, 0], commit[0, 0])
